# Initial kernel scaffold; baseline (speedup 1.0000x reference)
import functools

import jax
import jax.numpy as jnp
from jax.experimental import pallas as pl

NUM_EXPERTS = 64
TOP_K = 8
HIDDEN = 768

_NEG_INF = float("-inf")


def _router_block(x_ref, w_ref, b_ref, scores_ref, idx_ref):
    r = x_ref.shape[0]
    lt = jax.lax.dot_general(
        w_ref[...], x_ref[...],
        dimension_numbers=(((1,), (1,)), ((), ())),
        preferred_element_type=jnp.float32,
    ).astype(jnp.bfloat16)
    lt = lt + b_ref[...]

    eiota = jax.lax.broadcasted_iota(jnp.int32, (NUM_EXPERTS, r), 0)
    work = lt
    vals = []
    idxs = []
    for _ in range(TOP_K):
        m = jnp.max(work, axis=0, keepdims=True)
        cand = jnp.where(work == m, eiota, NUM_EXPERTS)
        sel = jnp.min(cand, axis=0, keepdims=True)
        vals.append(m)
        idxs.append(sel)
        work = jnp.where(eiota == sel, jnp.bfloat16(_NEG_INF), work)

    v = jnp.concatenate(vals, axis=0)
    e = jnp.exp((v - v[0:1]).astype(jnp.float32)).astype(jnp.bfloat16)
    s = jnp.sum(e, axis=0, keepdims=True)
    p = e / s

    acc = jnp.zeros((NUM_EXPERTS, r), dtype=jnp.bfloat16)
    for k in range(TOP_K):
        acc = jnp.where(eiota == idxs[k], p[k : k + 1], acc)

    scores_ref[...] = acc.T
    idx_ref[...] = jnp.concatenate(idxs, axis=0).T


@functools.partial(jax.jit, static_argnames=("block_r",))
def _router(x, w, b2d, block_r):
    n = x.shape[0]
    grid = (n // block_r,)
    return pl.pallas_call(
        _router_block,
        grid=grid,
        in_specs=[
            pl.BlockSpec((block_r, HIDDEN), lambda i: (i, 0)),
            pl.BlockSpec((NUM_EXPERTS, HIDDEN), lambda i: (0, 0)),
            pl.BlockSpec((NUM_EXPERTS, 1), lambda i: (0, 0)),
        ],
        out_specs=[
            pl.BlockSpec((block_r, NUM_EXPERTS), lambda i: (i, 0)),
            pl.BlockSpec((block_r, TOP_K), lambda i: (i, 0)),
        ],
        out_shape=[
            jax.ShapeDtypeStruct((n, NUM_EXPERTS), jnp.bfloat16),
            jax.ShapeDtypeStruct((n, TOP_K), jnp.int32),
        ],
    )(x, w, b2d)


def kernel(hidden_states, weight, bias):
    x = hidden_states.reshape(-1, HIDDEN)
    scores, idx = _router(x, weight, bias.reshape(NUM_EXPERTS, 1), 1024)
    return (scores, idx)


# speedup vs baseline: 10.4522x; 10.4522x over previous
import functools

import jax
import jax.numpy as jnp
from jax.experimental import pallas as pl

NUM_EXPERTS = 64
TOP_K = 8
HIDDEN = 768

_NEG_INF = float("-inf")
_TRUNC_MASK = -65536


def _router_block(x_ref, w_ref, b_ref, scores_ref, idx_ref):
    r = x_ref.shape[0]
    l32 = jax.lax.dot_general(
        x_ref[...], w_ref[...],
        dimension_numbers=(((1,), (1,)), ((), ())),
        preferred_element_type=jnp.float32,
    ) + b_ref[...]
    bits = jax.lax.bitcast_convert_type(l32, jnp.int32)
    logits = jax.lax.bitcast_convert_type(bits & _TRUNC_MASK, jnp.float32)

    shp = (r, NUM_EXPERTS)
    eiota = jax.lax.broadcasted_iota(jnp.int32, shp, 1)
    work = logits
    vals = []
    idxs = []
    hits = []
    for _ in range(TOP_K):
        m = jnp.max(work, axis=1, keepdims=True)
        mb = jnp.broadcast_to(m, shp)
        eqi = (work == mb).astype(jnp.int32)
        cand = eiota + (1 - eqi) * NUM_EXPERTS
        sel = jnp.min(cand, axis=1, keepdims=True)
        selb = jnp.broadcast_to(sel, shp)
        hit = cand == selb
        vals.append(m)
        idxs.append(sel)
        hits.append(hit)
        work = jnp.where(hit, jnp.float32(_NEG_INF), work)

    v = jnp.concatenate(vals, axis=1).astype(jnp.bfloat16)
    e = jnp.exp((v - jnp.broadcast_to(v[:, 0:1], v.shape)).astype(jnp.float32))
    e = e.astype(jnp.bfloat16)
    s = jnp.sum(e, axis=1, keepdims=True)
    p = e / jnp.broadcast_to(s, e.shape)

    acc = jnp.zeros(shp, dtype=jnp.bfloat16)
    for k in range(TOP_K):
        pb = jnp.broadcast_to(p[:, k : k + 1], shp)
        acc = acc + hits[k].astype(jnp.bfloat16) * pb

    scores_ref[...] = acc
    idx_ref[...] = jnp.concatenate(idxs, axis=1)


@functools.partial(jax.jit, static_argnames=("block_r",))
def _router(x, w, b2d, block_r):
    n = x.shape[0]
    grid = (n // block_r,)
    return pl.pallas_call(
        _router_block,
        grid=grid,
        in_specs=[
            pl.BlockSpec((block_r, HIDDEN), lambda i: (i, 0)),
            pl.BlockSpec((NUM_EXPERTS, HIDDEN), lambda i: (0, 0)),
            pl.BlockSpec((1, NUM_EXPERTS), lambda i: (0, 0)),
        ],
        out_specs=[
            pl.BlockSpec((block_r, NUM_EXPERTS), lambda i: (i, 0)),
            pl.BlockSpec((block_r, TOP_K), lambda i: (i, 0)),
        ],
        out_shape=[
            jax.ShapeDtypeStruct((n, NUM_EXPERTS), jnp.bfloat16),
            jax.ShapeDtypeStruct((n, TOP_K), jnp.int32),
        ],
    )(x, w, b2d)


def kernel(hidden_states, weight, bias):
    x = hidden_states.reshape(-1, HIDDEN)
    b2d = bias.astype(jnp.float32).reshape(1, NUM_EXPERTS)
    scores, idx = _router(x, weight, b2d, 1024)
    return (scores, idx)


# speedup vs baseline: 13.2440x; 1.2671x over previous
import functools

import jax
import jax.numpy as jnp
from jax.experimental import pallas as pl

NUM_EXPERTS = 64
TOP_K = 8
HIDDEN = 768

_HI16 = -65536
_POS = 0x7FFFFFFF
_NEGFLIP = 0x7FFF0000
_INT_MIN = -2147483648


def _router_block(x_ref, w_ref, b_ref, scores_ref, idx_ref):
    r = x_ref.shape[0]
    l32 = jax.lax.dot_general(
        x_ref[...], w_ref[...],
        dimension_numbers=(((1,), (1,)), ((), ())),
        preferred_element_type=jnp.float32,
    ) + b_ref[...]
    bits = jax.lax.bitcast_convert_type(l32, jnp.int32) & _HI16
    mono = bits ^ (jax.lax.shift_right_arithmetic(bits, 31) & _POS)

    shp = (r, NUM_EXPERTS)
    eiota = jax.lax.broadcasted_iota(jnp.int32, shp, 1)
    key = (mono & _HI16) | (NUM_EXPERTS - 1 - eiota)

    work = key
    keys = []
    hits = []
    for _ in range(TOP_K):
        m = jnp.max(work, axis=1, keepdims=True)
        mb = jnp.broadcast_to(m, shp)
        hit = work == mb
        keys.append(m)
        hits.append(hit)
        work = jnp.where(hit, _INT_MIN, work)

    kcat = jnp.concatenate(keys, axis=1)
    idx_ref[...] = (NUM_EXPERTS - 1) - (kcat & (NUM_EXPERTS - 1))

    vm = kcat & _HI16
    vbits = jnp.where(kcat < 0, vm ^ _NEGFLIP, vm)
    v = jax.lax.bitcast_convert_type(vbits, jnp.float32).astype(jnp.bfloat16)

    e = jnp.exp((v - jnp.broadcast_to(v[:, 0:1], v.shape)).astype(jnp.float32))
    e = e.astype(jnp.bfloat16)
    s = jnp.sum(e, axis=1, keepdims=True)
    p = e / jnp.broadcast_to(s, e.shape)

    acc = jnp.zeros(shp, dtype=jnp.bfloat16)
    for k in range(TOP_K):
        pb = jnp.broadcast_to(p[:, k : k + 1], shp)
        acc = acc + hits[k].astype(jnp.bfloat16) * pb
    scores_ref[...] = acc


@functools.partial(jax.jit, static_argnames=("block_r",))
def _router(x, w, b2d, block_r):
    n = x.shape[0]
    grid = (n // block_r,)
    return pl.pallas_call(
        _router_block,
        grid=grid,
        in_specs=[
            pl.BlockSpec((block_r, HIDDEN), lambda i: (i, 0)),
            pl.BlockSpec((NUM_EXPERTS, HIDDEN), lambda i: (0, 0)),
            pl.BlockSpec((1, NUM_EXPERTS), lambda i: (0, 0)),
        ],
        out_specs=[
            pl.BlockSpec((block_r, NUM_EXPERTS), lambda i: (i, 0)),
            pl.BlockSpec((block_r, TOP_K), lambda i: (i, 0)),
        ],
        out_shape=[
            jax.ShapeDtypeStruct((n, NUM_EXPERTS), jnp.bfloat16),
            jax.ShapeDtypeStruct((n, TOP_K), jnp.int32),
        ],
    )(x, w, b2d)


def kernel(hidden_states, weight, bias):
    x = hidden_states.reshape(-1, HIDDEN)
    b2d = bias.astype(jnp.float32).reshape(1, NUM_EXPERTS)
    scores, idx = _router(x, weight, b2d, 1024)
    return (scores, idx)


# speedup vs baseline: 13.7974x; 1.0418x over previous
import functools

import jax
import jax.numpy as jnp
from jax import lax
from jax.experimental import pallas as pl
from jax.experimental.pallas import tpu as pltpu
from jax.experimental.pallas import tpu_sc as plsc

NUM_EXPERTS = 64
TOP_K = 8
HIDDEN = 768
N_TOKENS = 32768

_HI16 = -65536
_POS = 0x7FFFFFFF
_NEGFLIP = 0x7FF0000 * 16 + 0xF0000

_info = plsc.get_sparse_core_info()
_NC, _NS = _info.num_cores, _info.num_subcores
_NW = _NC * _NS
_C = N_TOKENS // _NW
_H = 2
_CH = _C // _H
_G = _CH // 32


def _matmul_body(x_ref, w_ref, b_ref, kt_ref):
    r = x_ref.shape[0]
    l32 = jax.lax.dot_general(
        w_ref[...], x_ref[...],
        dimension_numbers=(((1,), (1,)), ((), ())),
        preferred_element_type=jnp.float32,
    ) + b_ref[...]
    bits = jax.lax.bitcast_convert_type(l32, jnp.int32) & _HI16
    mono = bits ^ (jax.lax.shift_right_arithmetic(bits, 31) & _POS)
    eiota = jax.lax.broadcasted_iota(jnp.int32, (NUM_EXPERTS, r), 0)
    kt_ref[...] = (mono & _HI16) | (NUM_EXPERTS - 1 - eiota)


@jax.jit
def _matmul_keys(x, w, b2d):
    n = x.shape[0]
    r = 2048
    return pl.pallas_call(
        _matmul_body,
        grid=(n // r,),
        in_specs=[
            pl.BlockSpec((r, HIDDEN), lambda i: (i, 0)),
            pl.BlockSpec((NUM_EXPERTS, HIDDEN), lambda i: (0, 0)),
            pl.BlockSpec((NUM_EXPERTS, 1), lambda i: (0, 0)),
        ],
        out_specs=pl.BlockSpec((NUM_EXPERTS, r), lambda i: (0, i)),
        out_shape=jax.ShapeDtypeStruct((NUM_EXPERTS, n), jnp.int32),
    )(x, w, b2d)


def _rtne(x16):
    b = jax.lax.bitcast_convert_type(x16, jnp.int32)
    r = b + 32767 + (jax.lax.shift_right_logical(b, 16) & 1)
    return jax.lax.bitcast_convert_type(r & _HI16, jnp.float32)


def _key_val(k):
    vm = k & _HI16
    vbits = jnp.where(k < 0, vm ^ _NEGFLIP, vm)
    return jax.lax.bitcast_convert_type(vbits, jnp.float32)


def _sc_body(kt_hbm, scores_hbm, idx_hbm, key_v, sc_v, idx_v, sem):
    wid = lax.axis_index("s") * _NC + lax.axis_index("c")
    lane = lax.iota(jnp.int32, 16)
    int_min = jnp.full((16,), -2147483647 - 1, jnp.int32)

    def _half(h, _):
        base = wid * _C + h * _CH
        pltpu.sync_copy(kt_hbm.at[:, pl.ds(base, _CH)], key_v)

        def _zero(t, c):
            sc_v[pl.ds(t * 16, 16)] = jnp.zeros((16,), jnp.float32)
            return c

        lax.fori_loop(0, _CH * NUM_EXPERTS // 16, _zero, 0)

        def _group(g, c):
            tops = []
            for sub in range(2):
                off = g * 32 + sub * 16
                ks = [int_min] * TOP_K
                for e in range(NUM_EXPERTS):
                    x = key_v[e, pl.ds(off, 16)]
                    for j in range(TOP_K):
                        cgt = x > ks[j]
                        nk = jnp.where(cgt, x, ks[j])
                        x = jnp.where(cgt, ks[j], x)
                        ks[j] = nk
                tops.append((off, ks))

            for off, ks in tops:
                tok = off + lane
                v = [_key_val(ks[j]) for j in range(TOP_K)]
                ei = [(NUM_EXPERTS - 1) - (ks[j] & (NUM_EXPERTS - 1))
                      for j in range(TOP_K)]
                d = [_rtne(v[j] - v[0]) for j in range(TOP_K)]
                e_ = [_rtne(jnp.exp(d[j])) for j in range(TOP_K)]
                s = e_[0]
                for j in range(1, TOP_K):
                    s = _rtne(s + e_[j])
                for j in range(TOP_K):
                    p = _rtne(e_[j] / s)
                    plsc.store_scatter(sc_v, [tok * NUM_EXPERTS + ei[j]], p)
                    plsc.store_scatter(idx_v, [tok * TOP_K + j], ei[j])
            return c

        lax.fori_loop(0, _G, _group, 0)

        pltpu.sync_copy(sc_v, scores_hbm.at[pl.ds(base * NUM_EXPERTS, _CH * NUM_EXPERTS)])
        pltpu.sync_copy(idx_v, idx_hbm.at[pl.ds(base * TOP_K, _CH * TOP_K)])
        return _

    lax.fori_loop(0, _H, _half, 0)


_sc_topk = pl.kernel(
    _sc_body,
    out_type=[
        jax.ShapeDtypeStruct((N_TOKENS * NUM_EXPERTS,), jnp.float32),
        jax.ShapeDtypeStruct((N_TOKENS * TOP_K,), jnp.int32),
    ],
    mesh=plsc.VectorSubcoreMesh(core_axis_name="c", subcore_axis_name="s"),
    compiler_params=pltpu.CompilerParams(
        needs_layout_passes=False, use_tc_tiling_on_sc=False),
    scratch_types=[
        pltpu.VMEM((NUM_EXPERTS, _CH), jnp.int32),
        pltpu.VMEM((_CH * NUM_EXPERTS,), jnp.float32),
        pltpu.VMEM((_CH * TOP_K,), jnp.int32),
        pltpu.SemaphoreType.DMA,
    ],
)


def kernel(hidden_states, weight, bias):
    x = hidden_states.reshape(-1, HIDDEN)
    b2d = bias.astype(jnp.float32).reshape(NUM_EXPERTS, 1)
    kt = _matmul_keys(x, weight, b2d)
    scores32, idx = _sc_topk(kt)
    return (scores32.reshape(N_TOKENS, NUM_EXPERTS).astype(jnp.bfloat16),
            idx.reshape(N_TOKENS, TOP_K))
